# baseline (speedup 1.0000x reference)
import functools

import jax
import jax.numpy as jnp
from jax import lax
from jax.experimental import pallas as pl
from jax.experimental.pallas import tpu as pltpu
from jax.experimental.pallas import tpu_sc as plsc

H = 7
NC, NS = 2, 16
NW = NC * NS
CHUNK = 128
KG = 8
ROW = 8
FOLD = 16
RB = 2048


def _sc_gather(ids3, table):
    nchunks = ids3.shape[1]
    mesh = plsc.VectorSubcoreMesh(core_axis_name="c", subcore_axis_name="s")

    @functools.partial(
        pl.kernel,
        out_type=jax.ShapeDtypeStruct((NW, nchunks, CHUNK, ROW), jnp.float32),
        mesh=mesh,
        scratch_types=[
            pltpu.VMEM((nchunks, CHUNK), jnp.int32),
            pltpu.VMEM((KG, CHUNK, ROW), jnp.float32),
            pltpu.SemaphoreType.DMA,
        ],
        compiler_params=pltpu.CompilerParams(use_tc_tiling_on_sc=False),
    )
    def gather_kernel(ids_hbm, table_hbm, out_hbm, idx_v, rows_v, sem):
        wid = lax.axis_index("s") * NC + lax.axis_index("c")
        pltpu.sync_copy(ids_hbm.at[wid], idx_v)

        def group(g, carry):
            copies = [
                pltpu.async_copy(
                    table_hbm.at[idx_v.at[g * KG + j]], rows_v.at[j], sem)
                for j in range(KG)
            ]
            for c in copies:
                c.wait()
            pltpu.sync_copy(rows_v, out_hbm.at[wid, pl.ds(g * KG, KG)])
            return carry

        lax.fori_loop(0, nchunks // KG, group, 0)

    return gather_kernel(ids3, table)


def _patterns(decay_slow, decay_fast):
    h = jnp.arange(1, H + 1, dtype=jnp.float32)
    inv_s = 1.0 / (h ** decay_slow)
    inv_f = 1.0 / (h ** decay_fast)
    r = jnp.arange(ROW * FOLD)[:, None]
    c = jnp.arange(2 * H * FOLD)[None, :]
    j = c // (2 * H)
    k = c % (2 * H)
    slow = k < H
    hk = jnp.where(slow, k, k - H)
    hval = jnp.take(h, hk)
    zero = jnp.float32(0.0)
    mf = (jnp.where((r == ROW * j) & slow, hval, zero)
          + jnp.where((r == ROW * j + 1) & ~slow, hval, zero))
    ma = (jnp.where((r == ROW * j + 2) & slow, jnp.take(inv_s, hk), zero)
          + jnp.where((r == ROW * j + 3) & ~slow, jnp.take(inv_f, hk), zero))
    mp = jnp.where(r == ROW * j + 4, jnp.float32(1.0), zero) + zero * c
    return mf, ma, mp.astype(jnp.float32)


def _tc_expand(g2, mf, ma, mp):
    rows, kdim = g2.shape
    n = mf.shape[1]

    def body(g_ref, mf_ref, ma_ref, mp_ref, of_ref, oa_ref, op_ref):
        g = g_ref[...]
        of_ref[...] = jnp.dot(g, mf_ref[...], preferred_element_type=jnp.float32)
        oa_ref[...] = jnp.dot(g, ma_ref[...], preferred_element_type=jnp.float32)
        op_ref[...] = jnp.dot(g, mp_ref[...], preferred_element_type=jnp.float32)

    const_spec = pl.BlockSpec((kdim, n), lambda i: (0, 0))
    return pl.pallas_call(
        body,
        grid=(rows // RB,),
        in_specs=[pl.BlockSpec((RB, kdim), lambda i: (i, 0)),
                  const_spec, const_spec, const_spec],
        out_specs=[pl.BlockSpec((RB, n), lambda i: (i, 0))] * 3,
        out_shape=[jax.ShapeDtypeStruct((rows, n), jnp.float32)] * 3,
    )(g2, mf, ma, mp)


def kernel(ids, freq_slow, freq_fast, amplitudes, phase, scale_mix,
           decay_slow, decay_fast):
    B, L = ids.shape
    T = B * L
    mix = jax.nn.sigmoid(scale_mix)
    mix_a = mix * amplitudes
    m1_a = (1.0 - mix) * amplitudes
    z = jnp.zeros_like(freq_slow)
    table = jnp.stack(
        [freq_slow, freq_fast, mix_a, m1_a, phase, z, z, z], axis=1)

    nchunks = T // (NW * CHUNK)
    ids3 = ids.reshape(NW, nchunks, CHUNK)
    g = _sc_gather(ids3, table)
    g2 = g.reshape(T // FOLD, ROW * FOLD)

    mf, ma, mp = _patterns(decay_slow, decay_fast)
    of, oa, op = _tc_expand(g2, mf, ma, mp)
    shape = (B, L, 2 * H)
    return of.reshape(shape), oa.reshape(shape), op.reshape(shape)
